# baseline (speedup 1.0000x reference)
import functools

import jax
import jax.numpy as jnp
from jax import lax
from jax.experimental import pallas as pl
from jax.experimental.pallas import tpu as pltpu
from jax.experimental.pallas import tpu_sc as plsc

_NC = 2
_NS = 16
_NW = _NC * _NS

_NSLAB = 25
_NSRC = 25
_NSTR = 31
_CAP = 8192
_Q = 512
_NROUND = _CAP // _Q
_CH = 128

_mesh = plsc.VectorSubcoreMesh(core_axis_name="c", subcore_axis_name="s")
_params = pltpu.CompilerParams(
    use_tc_tiling_on_sc=True, needs_layout_passes=False)


@functools.partial(
    pl.kernel,
    mesh=_mesh,
    out_type=(
        jax.ShapeDtypeStruct((_NSRC * _NSTR * _CAP,), jnp.int32),
        jax.ShapeDtypeStruct((_NSRC * _NSTR * _CAP,), jnp.int32),
        jax.ShapeDtypeStruct((_NSRC * 128,), jnp.int32),
    ),
    scratch_types=[
        pltpu.VMEM((8, 1024), jnp.int32),
        pltpu.VMEM((64, 128), jnp.int32),
        pltpu.VMEM((64, 128), jnp.int32),
        pltpu.VMEM((64, 128), jnp.int32),
        pltpu.VMEM((128,), jnp.int32),
        pltpu.SemaphoreType.DMA,
        pltpu.SemaphoreType.DMA,
    ],
    compiler_params=_params,
)
def _router(inp_hbm, hbi_hbm, hbj_hbm, cnt_hbm,
            slab, ivals, jvals, slots, cursor, dsem, ssem):
    u = lax.axis_index("s") * _NC + lax.axis_index("c")

    @pl.when(u < _NSRC)
    def _():
        pltpu.async_copy(inp_hbm.at[u], slab, dsem).wait()
        iota = lax.iota(jnp.int32, 16)
        jlane = iota * 200
        base_u = u * (_NSTR * _CAP)
        for z in range(8):
            base = lax.shift_left(iota + z * 16, 13) + base_u
            cursor[pl.ds(z * 16, 16)] = base
        for r in range(8):
            for c in range(64):
                v = slab[r, pl.ds(c * 16, 16)]
                jv = jlane + (c * 3200 + 8 * u + r)
                t = lax.shift_right_logical(v, 15)
                rank, last = plsc.scan_count(t)
                cur = plsc.load_gather(cursor, [t])
                gslot = jnp.clip(cur + rank, 0, _NSRC * _NSTR * _CAP - 1)
                k = r * 64 + c
                kr, kc = k // 8, (k % 8) * 16
                ivals[kr, pl.ds(kc, 16)] = v
                jvals[kr, pl.ds(kc, 16)] = jv
                slots[kr, pl.ds(kc, 16)] = gslot
                plsc.addupdate_scatter(cursor, [t], rank + 1, mask=last)
        cps = []
        for b in range(64):
            cps.append(pltpu.async_copy(
                ivals.at[b], hbi_hbm.at[slots.at[b]], ssem))
            cps.append(pltpu.async_copy(
                jvals.at[b], hbj_hbm.at[slots.at[b]], ssem))
        for cp in cps:
            cp.wait()
        for z in range(8):
            base = lax.shift_left(iota + z * 16, 13) + base_u
            cursor[pl.ds(z * 16, 16)] = cursor[pl.ds(z * 16, 16)] - base
        pltpu.sync_copy(cursor, cnt_hbm.at[pl.ds(u * 128, 128)])


@functools.partial(
    pl.kernel,
    mesh=_mesh,
    out_type=jax.ShapeDtypeStruct((204800, 128), jnp.float32),
    scratch_types=[
        pltpu.VMEM((64, _CH), jnp.float32),
        pltpu.VMEM((64, _CH), jnp.float32),
        pltpu.VMEM((_NSRC * _Q,), jnp.int32),
        pltpu.VMEM((_NSRC * _Q,), jnp.int32),
        pltpu.VMEM((_NSRC * _Q + 16,), jnp.int32),
        pltpu.VMEM((_NSRC * _Q + 16,), jnp.int32),
        pltpu.VMEM((256,), jnp.int32),
        pltpu.VMEM((288,), jnp.int32),
        pltpu.VMEM((256,), jnp.int32),
        pltpu.VMEM((_NSRC * 128,), jnp.int32),
        pltpu.VMEM((64, 128), jnp.float32),
        pltpu.VMEM((64,), jnp.int32),
        pltpu.VMEM((1, 128), jnp.float32),
        pltpu.VMEM((1,), jnp.int32),
        pltpu.SemaphoreType.DMA,
        pltpu.SemaphoreType.DMA,
        pltpu.SemaphoreType.DMA,
        pltpu.SemaphoreType.DMA,
        pltpu.SemaphoreType.DMA,
    ],
    compiler_params=_params,
)
def _serve(table_hbm, tail_hbm, hbi_hbm, hbj_hbm, cnt_hbm, out_hbm,
           slab0, slab1, seg_i, seg_j, bin_i, bin_j,
           hist, binst, cursor2, cnt_v,
           rowstage, rowj, row1, rowj1,
           segsem, s0sem, s1sem, fsem, csem):
    v = lax.axis_index("s") * _NC + lax.axis_index("c")

    def _sl(ref, i):
        return plsc.load_gather(ref, [jnp.full((16,), i, jnp.int32)])[0]

    @pl.when(v < _NSTR)
    def _():
        pltpu.async_copy(cnt_hbm, cnt_v, csem).wait()
        stripe_lo = v * 32768
        nchunks = jnp.where(v < 30, 256, 133)
        iota = lax.iota(jnp.int32, 16)
        zeros = jnp.zeros((16,), jnp.int32)
        ones = jnp.full((16,), 1, jnp.int32)
        m0 = iota == 0

        def _ss(ref, i, val):
            plsc.store_scatter(ref, [jnp.full((16,), i, jnp.int32)],
                               jnp.full((16,), val, jnp.int32), mask=m0)

        def _mx(u, m):
            return jnp.maximum(m, _sl(cnt_v, u * 128 + v))
        max_n = lax.fori_loop(0, _NSRC, _mx, jnp.int32(0))

        def round_body(r, rc):
            active = max_n > r * _Q

            def _tv(u, a):
                return a + jnp.clip(_sl(cnt_v, u * 128 + v) - r * _Q, 0, _Q)
            tot = lax.fori_loop(0, _NSRC, _tv, jnp.int32(0))

            @pl.when(active)
            def _():
                segs = []
                for u in range(_NSRC):
                    off = (u * _NSTR + v) * _CAP + r * _Q
                    segs.append(pltpu.async_copy(
                        hbi_hbm.at[pl.ds(off, _Q)],
                        seg_i.at[pl.ds(u * _Q, _Q)], segsem))
                    segs.append(pltpu.async_copy(
                        hbj_hbm.at[pl.ds(off, _Q)],
                        seg_j.at[pl.ds(u * _Q, _Q)], segsem))
                for cp in segs:
                    cp.wait()
                for z in range(16):
                    hist[pl.ds(z * 16, 16)] = zeros

                def hist_u(u, _):
                    nv = jnp.clip(_sl(cnt_v, u * 128 + v) - r * _Q, 0, _Q)

                    def hist_q(q, _):
                        iv = seg_i[pl.ds(u * _Q + q * 16, 16)]
                        m = (iota + q * 16) < nv
                        t = jnp.clip(
                            lax.shift_right_logical(iv - stripe_lo, 7), 0, 255)
                        plsc.addupdate_scatter(hist, [t], ones, mask=m)
                        return 0
                    return lax.fori_loop(0, _Q // 16, hist_q, 0)
                lax.fori_loop(0, _NSRC, hist_u, 0)

                def pfx(b, base):
                    h = hist[pl.ds(b * 16, 16)]
                    inc = plsc.cumsum(h)
                    excl = inc - h + base
                    binst[pl.ds(b * 16, 16)] = excl
                    cursor2[pl.ds(b * 16, 16)] = excl
                    return base + jnp.sum(h)
                lax.fori_loop(0, 16, pfx, jnp.int32(0))

                def place_u(u, _):
                    nv = jnp.clip(_sl(cnt_v, u * 128 + v) - r * _Q, 0, _Q)

                    def place_q(q, _):
                        iv = seg_i[pl.ds(u * _Q + q * 16, 16)]
                        jv = seg_j[pl.ds(u * _Q + q * 16, 16)]
                        m = (iota + q * 16) < nv
                        il = iv - stripe_lo
                        t = jnp.clip(lax.shift_right_logical(il, 7), 0, 255)
                        rank, last = plsc.scan_count(t, mask=m)
                        cur = plsc.load_gather(cursor2, [t])
                        slot = jnp.clip(cur + rank, 0, _NSRC * _Q + 15)
                        plsc.store_scatter(bin_i, [slot], il, mask=m)
                        plsc.store_scatter(bin_j, [slot], jv, mask=m)
                        plsc.addupdate_scatter(
                            cursor2, [t], rank + 1, mask=m & last)
                        return 0
                    return lax.fori_loop(0, _Q // 16, place_q, 0)
                lax.fori_loop(0, _NSRC, place_u, 0)

            def start_chunk(cc, sbuf, sem):
                @pl.when(active & (cc < nchunks))
                def _():
                    cbase = stripe_lo + cc * _CH

                    @pl.when(cbase < 999936)
                    def _():
                        pltpu.async_copy(
                            table_hbm.at[:, pl.ds(cbase, _CH)], sbuf, sem)

                    @pl.when(cbase >= 999936)
                    def _():
                        pltpu.async_copy(tail_hbm, sbuf, sem)

            def serve_chunk(cc, sbuf, sem, rc):
                ok = active & (cc < nchunks)

                @pl.when(ok)
                def _():
                    pltpu.make_async_copy(
                        table_hbm.at[:, pl.ds(0, _CH)], sbuf, sem).wait()
                ccl = jnp.minimum(cc, 255)
                gs = jnp.where(ok, _sl(binst, ccl), 0)
                ge = jnp.where(
                    ok,
                    jnp.where(cc < 255, _sl(binst, jnp.minimum(cc + 1, 255)),
                              tot), 0)

                def hit(h, rc):
                    il = _sl(bin_i, h)
                    lane = jnp.bitwise_and(il, _CH - 1)
                    lanes = jnp.full((16,), lane, jnp.int32)
                    rcv = jnp.full((16,), rc, jnp.int32)
                    for k in range(4):
                        dvec = iota + k * 16
                        vals = plsc.load_gather(sbuf, [dvec, lanes])
                        plsc.store_scatter(rowstage, [rcv, dvec], vals)
                    _ss(rowj, rc, jnp.clip(_sl(bin_j, h), 0, 204799))
                    rcn = rc + 1
                    full = rcn >= 64

                    @pl.when(full)
                    def _():
                        pltpu.async_copy(
                            rowstage, out_hbm.at[rowj],
                            fsem).wait()
                    return jnp.where(full, 0, rcn)
                return lax.fori_loop(gs, ge, hit, rc)

            start_chunk(jnp.int32(0), slab0, s0sem)
            start_chunk(jnp.int32(1), slab1, s1sem)

            def two(c2, rc):
                cc0 = c2 * 2
                rc = serve_chunk(cc0, slab0, s0sem, rc)
                start_chunk(cc0 + 2, slab0, s0sem)
                rc = serve_chunk(cc0 + 1, slab1, s1sem, rc)
                start_chunk(cc0 + 3, slab1, s1sem)
                return rc
            return lax.fori_loop(0, 128, two, rc)

        rc = lax.fori_loop(0, _NROUND, round_body, jnp.int32(0))

        def drain(h, _):
            rcv = jnp.zeros((16,), jnp.int32)
            hv = jnp.full((16,), h, jnp.int32)
            for k in range(4):
                dvec = iota + k * 16
                vals = plsc.load_gather(rowstage, [hv, dvec])
                plsc.store_scatter(row1, [rcv, dvec], vals)
            _ss(rowj1, 0, _sl(rowj, h))
            pltpu.async_copy(row1, out_hbm.at[rowj1],
                             fsem).wait()
            return 0
        lax.fori_loop(0, rc, drain, 0)


def kernel(inp, emb_weight):
    inp_t3 = inp.T.reshape(_NSLAB, 8, 1024)
    table_t = emb_weight.T
    tail = jnp.pad(emb_weight[999936:].T, ((0, 0), (0, 64)))
    hbi, hbj, cnt = _router(inp_t3)
    x = _serve(table_t, tail, hbi, hbj, cnt)
    return x.reshape(1024, 200, 128)[:, :, :64]


# speedup vs baseline: 7.4715x; 7.4715x over previous
import functools

import jax
import jax.numpy as jnp
from jax import lax
from jax.experimental import pallas as pl
from jax.experimental.pallas import tpu as pltpu
from jax.experimental.pallas import tpu_sc as plsc

EMB_DIM = 64
_NC = 2
_NS = 16
_NW = _NC * _NS


@functools.lru_cache(maxsize=None)
def _make_linearize(b, s):
    n_total = b * s
    n_slab = s // 8
    b_per_w = b // _NW
    j_per_w = n_total // _NW
    mesh = plsc.VectorSubcoreMesh(core_axis_name="c", subcore_axis_name="s")

    @functools.partial(
        pl.kernel,
        mesh=mesh,
        out_type=jax.ShapeDtypeStruct((n_total,), jnp.int32),
        scratch_types=[
            pltpu.VMEM((8, b), jnp.int32),
            pltpu.VMEM((8, b), jnp.int32),
            pltpu.VMEM((j_per_w,), jnp.int32),
            pltpu.SemaphoreType.DMA,
        ],
        compiler_params=pltpu.CompilerParams(
            use_tc_tiling_on_sc=True, needs_layout_passes=False),
    )
    def linearize_kernel(inp_hbm, out_hbm, slab0, slab1, stage, sem):
        wid = lax.axis_index("s") * _NC + lax.axis_index("c")
        b0 = wid * b_per_w
        slabs = [slab0, slab1]
        copies = [None] * n_slab
        copies[0] = pltpu.async_copy(inp_hbm.at[0], slabs[0], sem)
        if n_slab > 1:
            copies[1] = pltpu.async_copy(inp_hbm.at[1], slabs[1], sem)
        lane = lax.iota(jnp.int32, 16) * s
        for g in range(n_slab):
            copies[g].wait()
            slab = slabs[g % 2]
            for r in range(8):
                for c in range(b_per_w // 16):
                    v = slab[r, pl.ds(b0 + c * 16, 16)]
                    tgt = lane + (c * 16 * s + 8 * g + r)
                    plsc.store_scatter(stage, [tgt], v)
            if g + 2 < n_slab:
                copies[g + 2] = pltpu.async_copy(
                    inp_hbm.at[g + 2], slabs[g % 2], sem)
        pltpu.sync_copy(stage, out_hbm.at[pl.ds(wid * j_per_w, j_per_w)])

    return linearize_kernel


@functools.lru_cache(maxsize=None)
def _make_gather(n_total, chunk):
    b_per_w = n_total // _NW
    t = b_per_w // chunk
    mesh = plsc.VectorSubcoreMesh(core_axis_name="c", subcore_axis_name="s")

    @functools.partial(
        pl.kernel,
        mesh=mesh,
        out_type=jax.ShapeDtypeStruct((n_total, EMB_DIM), jnp.float32),
        scratch_types=[
            pltpu.VMEM((chunk,), jnp.int32),
            pltpu.VMEM((chunk,), jnp.int32),
            pltpu.VMEM((2, chunk, EMB_DIM), jnp.float32),
            pltpu.SemaphoreType.DMA,
            pltpu.SemaphoreType.DMA,
            pltpu.SemaphoreType.DMA,
        ],
        compiler_params=pltpu.CompilerParams(use_tc_tiling_on_sc=False),
    )
    def gather_kernel(idx_hbm, table_hbm, out_hbm,
                      idx_v0, idx_v1, rows_v, isem, gsem, ssem):
        wid = lax.axis_index("s") * _NC + lax.axis_index("c")
        idx_bufs = [idx_v0, idx_v1]
        ic = [None] * t
        gc = [None] * t
        sc = [None] * t

        def idx_slice(c):
            return idx_hbm.at[pl.ds((wid * t + c) * chunk, chunk)]

        ic[0] = pltpu.async_copy(idx_slice(0), idx_bufs[0], isem)
        if t > 1:
            ic[1] = pltpu.async_copy(idx_slice(1), idx_bufs[1], isem)
        ic[0].wait()
        gc[0] = pltpu.async_copy(table_hbm.at[idx_bufs[0]], rows_v.at[0], gsem)
        for c in range(t):
            p = c % 2
            if c + 1 < t:
                ic[c + 1].wait()
                if c >= 1:
                    sc[c - 1].wait()
                gc[c + 1] = pltpu.async_copy(
                    table_hbm.at[idx_bufs[(c + 1) % 2]],
                    rows_v.at[(c + 1) % 2], gsem)
            gc[c].wait()
            if c + 2 < t:
                ic[c + 2] = pltpu.async_copy(idx_slice(c + 2), idx_bufs[p], isem)
            sc[c] = pltpu.async_copy(
                rows_v.at[p],
                out_hbm.at[pl.ds((wid * t + c) * chunk, chunk)], ssem)
        if t > 1:
            sc[t - 2].wait()
        sc[t - 1].wait()

    return gather_kernel


def kernel(inp, emb_weight):
    b, s = inp.shape
    n_total = b * s
    inp_t3 = inp.T.reshape(s // 8, 8, b)
    idx_flat = _make_linearize(b, s)(inp_t3)
    out = _make_gather(n_total, 640)(idx_flat, emb_weight)
    return out.reshape(b, s, EMB_DIM)
